# Initial kernel scaffold; baseline (speedup 1.0000x reference)
import jax
import jax.numpy as jnp
from jax.experimental import pallas as pl


def kernel(hidden_states, W_gate, W1, b1, W2, b2):
    raise NotImplementedError("write your pallas kernel here")


# speedup vs baseline: 3.7329x; 3.7329x over previous
import jax
import jax.numpy as jnp
from jax.experimental import pallas as pl
from jax.experimental.pallas import tpu as pltpu

H = 768
E = 64
BASE_K = 8
MIN_K = 4
MAX_K = 12
AUX_W = 0.01
ENT_W = 0.001

T = 512
NEG = -3.0e38


def _router_block(x_ref, wg_ref, w1_ref, b1_ref, w2_ref, b2_ref,
                  w_out_ref, tpe_ref, ksum_ref, esum_ref):
    i = pl.program_id(0)
    x = x_ref[...]
    logits = jnp.dot(x, wg_ref[...], preferred_element_type=jnp.float32)

    h1_pre = jnp.dot(x, w1_ref[...], preferred_element_type=jnp.float32)
    h1_pre = h1_pre + b1_ref[...]
    h1 = h1_pre * jax.nn.sigmoid(h1_pre)
    d_pre = jnp.sum(h1 * w2_ref[...], axis=-1, keepdims=True) + b2_ref[...]
    difficulty = jax.nn.sigmoid(d_pre)
    k_float = MIN_K + difficulty * (MAX_K - MIN_K)
    k_int = jnp.clip(jnp.round(k_float), float(MIN_K), float(MAX_K))

    mx = jnp.max(logits, axis=-1, keepdims=True)
    thresh = mx
    l_cur = logits
    m = mx
    for rank in range(2, MAX_K + 1):
        l_cur = jnp.where(l_cur >= m, NEG, l_cur)
        m = jnp.max(l_cur, axis=-1, keepdims=True)
        if rank >= MIN_K:
            thresh = jnp.where(k_int == float(rank), m, thresh)

    mask = (logits >= thresh).astype(jnp.float32)
    e = jnp.exp(logits - mx) * mask
    s = jnp.sum(e, axis=-1, keepdims=True)
    w = e / s
    w_out_ref[...] = w

    d1 = difficulty
    ent = d1 * jnp.log(d1 + 1e-8) + (1.0 - d1) * jnp.log(1.0 - d1 + 1e-8)
    tpe_part = jnp.sum(w, axis=0, keepdims=True)
    ksum_part = jnp.sum(k_float).reshape(1, 1)
    esum_part = jnp.sum(ent).reshape(1, 1)

    @pl.when(i == 0)
    def _init():
        tpe_ref[...] = tpe_part
        ksum_ref[...] = ksum_part
        esum_ref[...] = esum_part

    @pl.when(i != 0)
    def _acc():
        tpe_ref[...] += tpe_part
        ksum_ref[...] += ksum_part
        esum_ref[...] += esum_part


def kernel(hidden_states, W_gate, W1, b1, W2, b2):
    B, S, _ = hidden_states.shape
    N = B * S
    x2d = hidden_states.reshape(N, H)
    wgT = W_gate.T
    w1T = W1.T
    Hq = W1.shape[0]
    b1r = b1.reshape(1, Hq)
    w2r = W2.reshape(1, Hq)
    b2r = b2.reshape(1, 1)

    grid = (N // T,)
    w_out, tpe, ksum, esum = pl.pallas_call(
        _router_block,
        grid=grid,
        in_specs=[
            pl.BlockSpec((T, H), lambda i: (i, 0)),
            pl.BlockSpec((H, E), lambda i: (0, 0)),
            pl.BlockSpec((H, Hq), lambda i: (0, 0)),
            pl.BlockSpec((1, Hq), lambda i: (0, 0)),
            pl.BlockSpec((1, Hq), lambda i: (0, 0)),
            pl.BlockSpec((1, 1), lambda i: (0, 0)),
        ],
        out_specs=[
            pl.BlockSpec((T, E), lambda i: (i, 0)),
            pl.BlockSpec((1, E), lambda i: (0, 0)),
            pl.BlockSpec((1, 1), lambda i: (0, 0)),
            pl.BlockSpec((1, 1), lambda i: (0, 0)),
        ],
        out_shape=[
            jax.ShapeDtypeStruct((N, E), jnp.float32),
            jax.ShapeDtypeStruct((1, E), jnp.float32),
            jax.ShapeDtypeStruct((1, 1), jnp.float32),
            jax.ShapeDtypeStruct((1, 1), jnp.float32),
        ],
        compiler_params=pltpu.CompilerParams(
            dimension_semantics=("arbitrary",),
        ),
    )(x2d, wgT, w1T, b1r, w2r, b2r)

    routing_weights = w_out.reshape(B, S, E)

    avg_k = ksum[0, 0] / N
    k_penalty = jax.nn.relu(BASE_K - avg_k) ** 2
    tpe_v = tpe[0]
    mean_tpe = jnp.mean(tpe_v)
    balance_loss = jnp.sum((tpe_v - mean_tpe) ** 2) / (E - 1) / (mean_tpe + 1e-8)
    entropy_bonus = esum[0, 0] / N
    aux_loss = AUX_W * (k_penalty + balance_loss) + ENT_W * entropy_bonus
    return routing_weights, aux_loss


# speedup vs baseline: 3.7439x; 1.0030x over previous
import jax
import jax.numpy as jnp
from jax.experimental import pallas as pl
from jax.experimental.pallas import tpu as pltpu

H = 768
E = 64
BASE_K = 8
MIN_K = 4
MAX_K = 12
AUX_W = 0.01
ENT_W = 0.001

T = 512
NEG = -3.0e38


def _router_block(x_ref, wg_ref, w1_ref, b1_ref, w2_ref, b2_ref,
                  w_out_ref, tpe_ref, ksum_ref, esum_ref):
    i = pl.program_id(0)
    x = x_ref[...]
    xb = x.astype(jnp.bfloat16)
    wgb = wg_ref[...].astype(jnp.bfloat16)
    w1b = w1_ref[...].astype(jnp.bfloat16)
    logits = jnp.dot(xb, wgb, preferred_element_type=jnp.float32)
    h1_pre = jnp.dot(xb, w1b, preferred_element_type=jnp.float32)
    h1_pre = h1_pre + b1_ref[...]
    h1 = h1_pre * jax.nn.sigmoid(h1_pre)
    h1b = h1.astype(jnp.bfloat16).astype(jnp.float32)
    w2b = w2_ref[...].astype(jnp.bfloat16).astype(jnp.float32)
    d_pre = jnp.sum(h1b * w2b, axis=-1, keepdims=True) + b2_ref[...]
    difficulty = jax.nn.sigmoid(d_pre)
    k_float = MIN_K + difficulty * (MAX_K - MIN_K)
    k_int = jnp.clip(jnp.round(k_float), float(MIN_K), float(MAX_K))

    mx = jnp.max(logits, axis=-1, keepdims=True)
    thresh = mx
    l_cur = logits
    m = mx
    for rank in range(2, MAX_K + 1):
        l_cur = jnp.where(l_cur >= m, NEG, l_cur)
        m = jnp.max(l_cur, axis=-1, keepdims=True)
        if rank >= MIN_K:
            thresh = jnp.where(k_int == float(rank), m, thresh)

    mask = (logits >= thresh).astype(jnp.float32)
    e = jnp.exp(logits - mx) * mask
    s = jnp.sum(e, axis=-1, keepdims=True)
    w = e / s
    w_out_ref[...] = w

    d1 = difficulty
    ent = d1 * jnp.log(d1 + 1e-8) + (1.0 - d1) * jnp.log(1.0 - d1 + 1e-8)
    tpe_part = jnp.sum(w, axis=0, keepdims=True)
    ksum_part = jnp.sum(k_float).reshape(1, 1)
    esum_part = jnp.sum(ent).reshape(1, 1)

    @pl.when(i == 0)
    def _init():
        tpe_ref[...] = tpe_part
        ksum_ref[...] = ksum_part
        esum_ref[...] = esum_part

    @pl.when(i != 0)
    def _acc():
        tpe_ref[...] += tpe_part
        ksum_ref[...] += ksum_part
        esum_ref[...] += esum_part


def kernel(hidden_states, W_gate, W1, b1, W2, b2):
    B, S, _ = hidden_states.shape
    N = B * S
    x2d = hidden_states.reshape(N, H)
    wgT = W_gate.T
    w1T = W1.T
    Hq = W1.shape[0]
    b1r = b1.reshape(1, Hq)
    w2r = W2.reshape(1, Hq)
    b2r = b2.reshape(1, 1)

    grid = (N // T,)
    w_out, tpe, ksum, esum = pl.pallas_call(
        _router_block,
        grid=grid,
        in_specs=[
            pl.BlockSpec((T, H), lambda i: (i, 0)),
            pl.BlockSpec((H, E), lambda i: (0, 0)),
            pl.BlockSpec((H, Hq), lambda i: (0, 0)),
            pl.BlockSpec((1, Hq), lambda i: (0, 0)),
            pl.BlockSpec((1, Hq), lambda i: (0, 0)),
            pl.BlockSpec((1, 1), lambda i: (0, 0)),
        ],
        out_specs=[
            pl.BlockSpec((T, E), lambda i: (i, 0)),
            pl.BlockSpec((1, E), lambda i: (0, 0)),
            pl.BlockSpec((1, 1), lambda i: (0, 0)),
            pl.BlockSpec((1, 1), lambda i: (0, 0)),
        ],
        out_shape=[
            jax.ShapeDtypeStruct((N, E), jnp.float32),
            jax.ShapeDtypeStruct((1, E), jnp.float32),
            jax.ShapeDtypeStruct((1, 1), jnp.float32),
            jax.ShapeDtypeStruct((1, 1), jnp.float32),
        ],
        compiler_params=pltpu.CompilerParams(
            dimension_semantics=("arbitrary",),
        ),
    )(x2d, wgT, w1T, b1r, w2r, b2r)

    routing_weights = w_out.reshape(B, S, E)

    avg_k = ksum[0, 0] / N
    k_penalty = jax.nn.relu(BASE_K - avg_k) ** 2
    tpe_v = tpe[0]
    mean_tpe = jnp.mean(tpe_v)
    balance_loss = jnp.sum((tpe_v - mean_tpe) ** 2) / (E - 1) / (mean_tpe + 1e-8)
    entropy_bonus = esum[0, 0] / N
    aux_loss = AUX_W * (k_penalty + balance_loss) + ENT_W * entropy_bonus
    return routing_weights, aux_loss
